# baseline (speedup 1.0000x reference)
import functools

import jax
import jax.numpy as jnp
from jax import lax
from jax.experimental import pallas as pl
from jax.experimental.pallas import tpu as pltpu
from jax.experimental.pallas import tpu_sc as plsc

BATCH = 16384
HIST = 200
VOCAB = 500
DIM = 12

NCORES = 2
NSUB = 16
NWORK = NCORES * NSUB
LANES = 16

VPAD = 512
NPAIR = DIM // 2
WSTRIDE = 7

BAGS_PER_W = BATCH // NWORK
HALF = BAGS_PER_W // 2
NBLK_H = HALF // LANES
UNROLL = 8


@functools.partial(
    pl.kernel,
    out_type=jax.ShapeDtypeStruct((DIM, BATCH), jnp.float32),
    mesh=plsc.VectorSubcoreMesh(core_axis_name="c", subcore_axis_name="s"),
    compiler_params=pltpu.CompilerParams(needs_layout_passes=False),
    scratch_types=[
        pltpu.VMEM((VPAD * WSTRIDE,), jnp.int32),
        pltpu.VMEM((HIST, HALF), jnp.int32),
        pltpu.VMEM((HIST, HALF), jnp.int32),
        pltpu.VMEM((DIM, BAGS_PER_W), jnp.float32),
        pltpu.SemaphoreType.DMA,
        pltpu.SemaphoreType.DMA,
    ],
)
def _embbag_kernel(xt_hbm, w_hbm, out_hbm, w_v, x_a, x_b, o_v, sem_a, sem_b):
    wid = lax.axis_index("s") * NCORES + lax.axis_index("c")
    wbase = wid * BAGS_PER_W
    cp_a = pltpu.make_async_copy(xt_hbm.at[:, pl.ds(wbase, HALF)], x_a, sem_a)
    cp_b = pltpu.make_async_copy(
        xt_hbm.at[:, pl.ds(wbase + HALF, HALF)], x_b, sem_b
    )
    cp_a.start()
    cp_b.start()
    pltpu.sync_copy(w_hbm, w_v)

    inv_l = jnp.float32(1.0 / HIST)
    zero = jnp.zeros((LANES,), jnp.float32)

    def make_block_body(x_v, o_off):
        def block_body(blk, carry):
            b0 = blk * LANES

            def l_body(j, accs):
                l0 = j * UNROLL
                new = list(accs)
                for u in range(UNROLL):
                    xv = x_v[l0 + u, pl.ds(b0, LANES)]
                    pos = xv * WSTRIDE
                    for k in range(NPAIR):
                        wk = plsc.load_gather(w_v, [pos + k])
                        new[2 * k] = new[2 * k] + plsc.bitcast(
                            lax.shift_left(wk, 16), jnp.float32
                        )
                        new[2 * k + 1] = new[2 * k + 1] + plsc.bitcast(
                            wk, jnp.float32
                        )
                return tuple(new)

            accs = lax.fori_loop(0, HIST // UNROLL, l_body, (zero,) * DIM)
            for d in range(DIM):
                o_v[d, pl.ds(o_off + b0, LANES)] = accs[d] * inv_l
            return carry

        return block_body

    cp_a.wait()
    lax.fori_loop(0, NBLK_H, make_block_body(x_a, 0), 0)
    cp_b.wait()
    lax.fori_loop(0, NBLK_H, make_block_body(x_b, HALF), 0)
    pltpu.sync_copy(o_v, out_hbm.at[:, pl.ds(wbase, BAGS_PER_W)])


def kernel(x_user, weight):
    wb = jax.lax.bitcast_convert_type(
        weight.astype(jnp.bfloat16), jnp.uint16
    ).astype(jnp.uint32)
    packed = (wb[:, 0::2] | (wb[:, 1::2] << 16)).astype(jnp.int32)
    wpad = (
        jnp.zeros((VPAD, WSTRIDE), jnp.int32)
        .at[:VOCAB, :NPAIR]
        .set(packed)
        .reshape(-1)
    )
    out_t = _embbag_kernel(x_user.astype(jnp.int32).T, wpad)
    return out_t.T


# speedup vs baseline: 1.0367x; 1.0367x over previous
import functools

import jax
import jax.numpy as jnp
from jax import lax
from jax.experimental import pallas as pl
from jax.experimental.pallas import tpu as pltpu
from jax.experimental.pallas import tpu_sc as plsc

BATCH = 16384
HIST = 200
VOCAB = 500
DIM = 12

NCORES = 2
NSUB = 16
NWORK = NCORES * NSUB
LANES = 16

VPAD = 512
NPAIR = DIM // 2
WSTRIDE = 7

BAGS_PER_W = BATCH // NWORK
HALF = BAGS_PER_W // 2
NBLK_H = HALF // LANES
UNROLL = 4


@functools.partial(
    pl.kernel,
    out_type=jax.ShapeDtypeStruct((DIM, BATCH), jnp.float32),
    mesh=plsc.VectorSubcoreMesh(core_axis_name="c", subcore_axis_name="s"),
    compiler_params=pltpu.CompilerParams(needs_layout_passes=False),
    scratch_types=[
        pltpu.VMEM((VPAD * WSTRIDE,), jnp.int32),
        pltpu.VMEM((HIST, HALF), jnp.int32),
        pltpu.VMEM((HIST, HALF), jnp.int32),
        pltpu.VMEM((DIM, BAGS_PER_W), jnp.float32),
        pltpu.SemaphoreType.DMA,
        pltpu.SemaphoreType.DMA,
    ],
)
def _embbag_kernel(xt_hbm, w_hbm, out_hbm, w_v, x_a, x_b, o_v, sem_a, sem_b):
    wid = lax.axis_index("s") * NCORES + lax.axis_index("c")
    wbase = wid * BAGS_PER_W
    cp_a = pltpu.make_async_copy(xt_hbm.at[:, pl.ds(wbase, HALF)], x_a, sem_a)
    cp_b = pltpu.make_async_copy(
        xt_hbm.at[:, pl.ds(wbase + HALF, HALF)], x_b, sem_b
    )
    cp_a.start()
    cp_b.start()
    pltpu.sync_copy(w_hbm, w_v)

    inv_l = jnp.float32(1.0 / HIST)
    zero = jnp.zeros((LANES,), jnp.float32)

    def make_block_body(x_v, o_off):
        def block_body(blk, carry):
            b0 = blk * LANES

            def l_body(j, accs):
                l0 = j * UNROLL
                new = list(accs)
                for u in range(UNROLL):
                    xv = x_v[l0 + u, pl.ds(b0, LANES)]
                    pos = xv * WSTRIDE
                    for k in range(NPAIR):
                        wk = plsc.load_gather(w_v, [pos + k])
                        new[2 * k] = new[2 * k] + plsc.bitcast(
                            lax.shift_left(wk, 16), jnp.float32
                        )
                        new[2 * k + 1] = new[2 * k + 1] + plsc.bitcast(
                            wk, jnp.float32
                        )
                return tuple(new)

            accs = lax.fori_loop(0, HIST // UNROLL, l_body, (zero,) * DIM)
            for d in range(DIM):
                o_v[d, pl.ds(o_off + b0, LANES)] = accs[d] * inv_l
            return carry

        return block_body

    cp_a.wait()
    lax.fori_loop(0, NBLK_H, make_block_body(x_a, 0), 0)
    cp_b.wait()
    lax.fori_loop(0, NBLK_H, make_block_body(x_b, HALF), 0)
    pltpu.sync_copy(o_v, out_hbm.at[:, pl.ds(wbase, BAGS_PER_W)])


def kernel(x_user, weight):
    wb = jax.lax.bitcast_convert_type(
        weight.astype(jnp.bfloat16), jnp.uint16
    ).astype(jnp.uint32)
    packed = (wb[:, 0::2] | (wb[:, 1::2] << 16)).astype(jnp.int32)
    wpad = (
        jnp.zeros((VPAD, WSTRIDE), jnp.int32)
        .at[:VOCAB, :NPAIR]
        .set(packed)
        .reshape(-1)
    )
    out_t = _embbag_kernel(x_user.astype(jnp.int32).T, wpad)
    return out_t.T


# speedup vs baseline: 1.1408x; 1.1005x over previous
import functools

import jax
import jax.numpy as jnp
from jax import lax
from jax.experimental import pallas as pl
from jax.experimental.pallas import tpu as pltpu
from jax.experimental.pallas import tpu_sc as plsc

BATCH = 16384
HIST = 200
VOCAB = 500
DIM = 12

NCORES = 2
NSUB = 16
NWORK = NCORES * NSUB
LANES = 16

VPAD = 512
NPAIR = DIM // 2
ROWW = NPAIR * LANES

BAGS_PER_W = BATCH // NWORK
CHUNK = 128
NCHUNK = BAGS_PER_W // CHUNK
NBLK_C = CHUNK // LANES
UNROLL = 4


@functools.partial(
    pl.kernel,
    out_type=jax.ShapeDtypeStruct((DIM, BATCH), jnp.float32),
    mesh=plsc.VectorSubcoreMesh(core_axis_name="c", subcore_axis_name="s"),
    compiler_params=pltpu.CompilerParams(needs_layout_passes=False),
    scratch_types=[
        pltpu.VMEM((VPAD * ROWW,), jnp.int32),
        pltpu.VMEM((HIST, CHUNK), jnp.int32),
        pltpu.VMEM((HIST, CHUNK), jnp.int32),
        pltpu.VMEM((DIM, BAGS_PER_W), jnp.float32),
        pltpu.SemaphoreType.DMA,
        pltpu.SemaphoreType.DMA,
        pltpu.SemaphoreType.DMA,
        pltpu.SemaphoreType.DMA,
    ],
)
def _embbag_kernel(xt_hbm, w_hbm, out_hbm, w_v, x_a, x_b, o_v, *sems):
    wid = lax.axis_index("s") * NCORES + lax.axis_index("c")
    wbase = wid * BAGS_PER_W
    bufs = (x_a, x_b)
    cps = [
        pltpu.make_async_copy(
            xt_hbm.at[:, pl.ds(wbase + c * CHUNK, CHUNK)], bufs[c % 2], sems[c]
        )
        for c in range(NCHUNK)
    ]
    cps[0].start()
    cps[1].start()
    pltpu.sync_copy(w_hbm, w_v)

    iota = lax.broadcasted_iota(jnp.int32, (LANES,), 0)
    inv_l = jnp.float32(1.0 / HIST)
    zero = jnp.zeros((LANES,), jnp.float32)
    wk_views = [w_v.at[pl.ds(k * LANES, VPAD * ROWW - NPAIR * LANES)]
                for k in range(NPAIR)]

    def make_block_body(x_v, o_off):
        def block_body(blk, carry):
            b0 = blk * LANES

            def l_body(j, accs):
                l0 = j * UNROLL
                new = list(accs)
                for u in range(UNROLL):
                    xv = x_v[l0 + u, pl.ds(b0, LANES)]
                    pos = xv * ROWW + iota
                    for k in range(NPAIR):
                        wk = plsc.load_gather(wk_views[k], [pos])
                        new[2 * k] = new[2 * k] + plsc.bitcast(
                            lax.shift_left(wk, 16), jnp.float32
                        )
                        new[2 * k + 1] = new[2 * k + 1] + plsc.bitcast(
                            wk, jnp.float32
                        )
                return tuple(new)

            accs = lax.fori_loop(0, HIST // UNROLL, l_body, (zero,) * DIM)
            for d in range(DIM):
                o_v[d, pl.ds(o_off + b0, LANES)] = accs[d] * inv_l
            return carry

        return block_body

    for c in range(NCHUNK):
        cps[c].wait()
        lax.fori_loop(0, NBLK_C, make_block_body(bufs[c % 2], c * CHUNK), 0)
        if c + 2 < NCHUNK:
            cps[c + 2].start()
    pltpu.sync_copy(o_v, out_hbm.at[:, pl.ds(wbase, BAGS_PER_W)])


def kernel(x_user, weight):
    wb = jax.lax.bitcast_convert_type(
        weight.astype(jnp.bfloat16), jnp.uint16
    ).astype(jnp.uint32)
    packed = (wb[:, 0::2] | (wb[:, 1::2] << 16)).astype(jnp.int32)
    wrep = jnp.zeros((VPAD, NPAIR, LANES), jnp.int32).at[:VOCAB].set(
        jnp.broadcast_to(packed[:, :, None].astype(jnp.int32),
                         (VOCAB, NPAIR, LANES))
    )
    out_t = _embbag_kernel(x_user.astype(jnp.int32).T, wrep.reshape(-1))
    return out_t.T


# speedup vs baseline: 1.1414x; 1.0005x over previous
import functools

import jax
import jax.numpy as jnp
from jax import lax
from jax.experimental import pallas as pl
from jax.experimental.pallas import tpu as pltpu
from jax.experimental.pallas import tpu_sc as plsc

BATCH = 16384
HIST = 200
VOCAB = 500
DIM = 12

NCORES = 2
NSUB = 16
NWORK = NCORES * NSUB
LANES = 16

VPAD = 512
NPAIR = DIM // 2
ROWW = NPAIR * LANES

BAGS_PER_W = BATCH // NWORK
CHUNK = 128
NCHUNK = BAGS_PER_W // CHUNK
NBLK_C = CHUNK // LANES
UNROLL = 4


@functools.partial(
    pl.kernel,
    out_type=jax.ShapeDtypeStruct((DIM, BATCH), jnp.float32),
    mesh=plsc.VectorSubcoreMesh(core_axis_name="c", subcore_axis_name="s"),
    compiler_params=pltpu.CompilerParams(needs_layout_passes=False),
    scratch_types=[
        pltpu.VMEM((VPAD * ROWW,), jnp.int32),
        pltpu.VMEM((HIST, CHUNK), jnp.int32),
        pltpu.VMEM((HIST, CHUNK), jnp.int32),
        pltpu.VMEM((DIM, BAGS_PER_W), jnp.float32),
        pltpu.SemaphoreType.DMA,
        pltpu.SemaphoreType.DMA,
        pltpu.SemaphoreType.DMA,
        pltpu.SemaphoreType.DMA,
    ],
)
def _embbag_kernel(xt_hbm, w_hbm, out_hbm, w_v, x_a, x_b, o_v, *sems):
    wid = lax.axis_index("s") * NCORES + lax.axis_index("c")
    wbase = wid * BAGS_PER_W
    bufs = (x_a, x_b)
    cps = [
        pltpu.make_async_copy(
            xt_hbm.at[:, pl.ds(wbase + c * CHUNK, CHUNK)], bufs[c % 2], sems[c]
        )
        for c in range(NCHUNK)
    ]
    cps[0].start()
    cps[1].start()
    pltpu.sync_copy(w_hbm, w_v)

    iota = lax.broadcasted_iota(jnp.int32, (LANES,), 0)
    inv_l = jnp.float32(1.0 / HIST)
    zero = jnp.zeros((LANES,), jnp.float32)
    wk_views = [w_v.at[pl.ds(k * LANES, VPAD * ROWW - NPAIR * LANES)]
                for k in range(NPAIR)]

    def make_pair_body(x_v, o_off):
        def pair_body(bp, carry):
            b0 = bp * (2 * LANES)

            def l_body(j, accs):
                l0 = j * UNROLL
                new = list(accs)
                for u in range(UNROLL):
                    row = l0 + u
                    for h in range(2):
                        xv = x_v[row, pl.ds(b0 + h * LANES, LANES)]
                        pos = xv * ROWW + iota
                        for k in range(NPAIR):
                            wk = plsc.load_gather(wk_views[k], [pos])
                            a = 12 * h + 2 * k
                            new[a] = new[a] + plsc.bitcast(
                                lax.shift_left(wk, 16), jnp.float32
                            )
                            new[a + 1] = new[a + 1] + plsc.bitcast(
                                wk, jnp.float32
                            )
                return tuple(new)

            accs = lax.fori_loop(0, HIST // UNROLL, l_body, (zero,) * (2 * DIM))
            for h in range(2):
                for d in range(DIM):
                    o_v[d, pl.ds(o_off + b0 + h * LANES, LANES)] = (
                        accs[12 * h + d] * inv_l
                    )
            return carry

        return pair_body

    for c in range(NCHUNK):
        cps[c].wait()
        lax.fori_loop(0, NBLK_C // 2, make_pair_body(bufs[c % 2], c * CHUNK), 0)
        if c + 2 < NCHUNK:
            cps[c + 2].start()
    pltpu.sync_copy(o_v, out_hbm.at[:, pl.ds(wbase, BAGS_PER_W)])


def kernel(x_user, weight):
    wb = jax.lax.bitcast_convert_type(
        weight.astype(jnp.bfloat16), jnp.uint16
    ).astype(jnp.uint32)
    packed = (wb[:, 0::2] | (wb[:, 1::2] << 16)).astype(jnp.int32)
    wrep = jnp.zeros((VPAD, NPAIR, LANES), jnp.int32).at[:VOCAB].set(
        jnp.broadcast_to(packed[:, :, None].astype(jnp.int32),
                         (VOCAB, NPAIR, LANES))
    )
    out_t = _embbag_kernel(x_user.astype(jnp.int32).T, wrep.reshape(-1))
    return out_t.T


# speedup vs baseline: 1.1919x; 1.0443x over previous
import functools

import jax
import jax.numpy as jnp
from jax import lax
from jax.experimental import pallas as pl
from jax.experimental.pallas import tpu as pltpu
from jax.experimental.pallas import tpu_sc as plsc

BATCH = 16384
HIST = 200
VOCAB = 500
DIM = 12

NCORES = 2
NSUB = 16
NWORK = NCORES * NSUB
LANES = 16

VPAD = 512
NPAIR = DIM // 2
ROWW = NPAIR * LANES

BAGS_PER_W = BATCH // NWORK
CHUNK = 128
NCHUNK = BAGS_PER_W // CHUNK
NBLK_C = CHUNK // LANES
UNROLL = 2


@functools.partial(
    pl.kernel,
    out_type=jax.ShapeDtypeStruct((DIM, BATCH), jnp.float32),
    mesh=plsc.VectorSubcoreMesh(core_axis_name="c", subcore_axis_name="s"),
    compiler_params=pltpu.CompilerParams(needs_layout_passes=False),
    scratch_types=[
        pltpu.VMEM((VPAD * ROWW,), jnp.int32),
        pltpu.VMEM((HIST, CHUNK), jnp.int32),
        pltpu.VMEM((HIST, CHUNK), jnp.int32),
        pltpu.VMEM((DIM, BAGS_PER_W), jnp.float32),
        pltpu.SemaphoreType.DMA,
        pltpu.SemaphoreType.DMA,
        pltpu.SemaphoreType.DMA,
        pltpu.SemaphoreType.DMA,
    ],
)
def _embbag_kernel(xt_hbm, w_hbm, out_hbm, w_v, x_a, x_b, o_v, *sems):
    wid = lax.axis_index("s") * NCORES + lax.axis_index("c")
    wbase = wid * BAGS_PER_W
    bufs = (x_a, x_b)
    cps = [
        pltpu.make_async_copy(
            xt_hbm.at[:, pl.ds(wbase + c * CHUNK, CHUNK)], bufs[c % 2], sems[c]
        )
        for c in range(NCHUNK)
    ]
    cps[0].start()
    cps[1].start()
    pltpu.sync_copy(w_hbm, w_v)

    iota = lax.broadcasted_iota(jnp.int32, (LANES,), 0)
    inv_l = jnp.float32(1.0 / HIST)
    zero = jnp.zeros((LANES,), jnp.float32)
    wk_views = [w_v.at[pl.ds(k * LANES, VPAD * ROWW - NPAIR * LANES)]
                for k in range(NPAIR)]

    def make_pair_body(x_v, o_off):
        def pair_body(bp, carry):
            b0 = bp * (2 * LANES)

            def l_body(j, accs):
                l0 = j * UNROLL
                new = list(accs)
                for u in range(UNROLL):
                    row = l0 + u
                    for h in range(2):
                        xv = x_v[row, pl.ds(b0 + h * LANES, LANES)]
                        pos = xv * ROWW + iota
                        for k in range(NPAIR):
                            wk = plsc.load_gather(wk_views[k], [pos])
                            a = 12 * h + 2 * k
                            new[a] = new[a] + plsc.bitcast(
                                lax.shift_left(wk, 16), jnp.float32
                            )
                            new[a + 1] = new[a + 1] + plsc.bitcast(
                                wk, jnp.float32
                            )
                return tuple(new)

            accs = lax.fori_loop(0, HIST // UNROLL, l_body, (zero,) * (2 * DIM))
            for h in range(2):
                for d in range(DIM):
                    o_v[d, pl.ds(o_off + b0 + h * LANES, LANES)] = (
                        accs[12 * h + d] * inv_l
                    )
            return carry

        return pair_body

    for c in range(NCHUNK):
        cps[c].wait()
        lax.fori_loop(0, NBLK_C // 2, make_pair_body(bufs[c % 2], c * CHUNK), 0)
        if c + 2 < NCHUNK:
            cps[c + 2].start()
    pltpu.sync_copy(o_v, out_hbm.at[:, pl.ds(wbase, BAGS_PER_W)])


def kernel(x_user, weight):
    wb = jax.lax.bitcast_convert_type(
        weight.astype(jnp.bfloat16), jnp.uint16
    ).astype(jnp.uint32)
    packed = (wb[:, 0::2] | (wb[:, 1::2] << 16)).astype(jnp.int32)
    wrep = jnp.zeros((VPAD, NPAIR, LANES), jnp.int32).at[:VOCAB].set(
        jnp.broadcast_to(packed[:, :, None].astype(jnp.int32),
                         (VOCAB, NPAIR, LANES))
    )
    out_t = _embbag_kernel(x_user.astype(jnp.int32).T, wrep.reshape(-1))
    return out_t.T
